# baseline (speedup 1.0000x reference)
import functools

import jax
import jax.numpy as jnp
from jax import lax
from jax.experimental import pallas as pl
from jax.experimental.pallas import tpu as pltpu
from jax.experimental.pallas import tpu_sc as plsc

_INPUT_BITS = 4096
_NUM_SUB = 4
_BITS_PER_SUB = 12
_TABLE = 1 << _BITS_PER_SUB
_NW = 32
_JPW = _INPUT_BITS // _NW
_L = 16
_NCH = _JPW // _L
_LINE = 128
_P = 8


def _sc_body(x_hbm, conn_hbm, tab_hbm, out_hbm,
             x_v, conn_v, line_v, out_v, *sems):
    wid = lax.axis_index("s") * 2 + lax.axis_index("c")
    base = wid * _JPW

    pltpu.sync_copy(x_hbm, x_v)
    pltpu.sync_copy(conn_hbm.at[:, :, pl.ds(base, _JPW)], conn_v)

    tab2 = tab_hbm.reshape(_NUM_SUB * _INPUT_BITS, _TABLE)
    lane = lax.iota(jnp.int32, _L)
    bvecs = [jnp.full((_L,), b, jnp.int32) for b in range(_BITS_PER_SUB)]
    kmasks = [lane == k for k in range(_L)]

    ones = [jnp.zeros((_L,), jnp.int32) for _ in range(_NCH)]
    pending = []

    def fire(jc, i):
        jloc = jc * _L + lane
        ivec = jnp.full((_L,), i, jnp.int32)
        addr = jnp.zeros((_L,), jnp.int32)
        for b in range(_BITS_PER_SUB):
            bits = plsc.load_gather(conn_v, [bvecs[b], ivec, jloc])
            bit = plsc.load_gather(x_v, [bits])
            addr = addr + (bit << b)
        ahi = addr >> 7
        slot = (jc * _NUM_SUB + i) % _P
        copies = []
        for k in range(_L):
            ahi_k = jnp.max(jnp.where(kmasks[k], ahi, 0))
            row_k = base + (i * _INPUT_BITS + jc * _L + k)
            copies.append(pltpu.async_copy(
                tab2.at[pl.ds(row_k, 1), pl.ds(ahi_k * _LINE, _LINE)],
                line_v.at[pl.ds(slot * _L + k, 1), :], sems[slot]))
        pending.append((jc, i, addr, copies))

    def drain():
        jc, i, addr, copies = pending.pop(0)
        for c in copies:
            c.wait()
        slot = (jc * _NUM_SUB + i) % _P
        v = plsc.load_gather(
            line_v, [slot * _L + lane, addr & (_LINE - 1)])
        ones[jc] = ones[jc] + jnp.where(v > 0.5, 1, 0).astype(jnp.int32)

    for c in range(_NCH * _NUM_SUB):
        fire(c // _NUM_SUB, c % _NUM_SUB)
        if len(pending) == _P:
            drain()
    while pending:
        drain()

    for jc in range(_NCH):
        out_v[pl.ds(jc * _L, _L)] = (
            jnp.where(ones[jc] > 2, 1, 0).astype(jnp.int32))

    pltpu.sync_copy(out_v, out_hbm.at[pl.ds(base, _JPW)])


@functools.partial(
    pl.kernel,
    out_type=jax.ShapeDtypeStruct((_INPUT_BITS,), jnp.int32),
    mesh=plsc.VectorSubcoreMesh(core_axis_name="c", subcore_axis_name="s"),
    compiler_params=pltpu.CompilerParams(
        needs_layout_passes=False, skip_device_barrier=True),
    scratch_types=(
        [pltpu.VMEM((_INPUT_BITS,), jnp.int32),
         pltpu.VMEM((_BITS_PER_SUB, _NUM_SUB, _JPW), jnp.int32),
         pltpu.VMEM((_P * _L, _LINE), jnp.float32),
         pltpu.VMEM((_JPW,), jnp.int32)]
        + [pltpu.SemaphoreType.DMA] * _P),
)
def _sc_kernel(x_hbm, conn_hbm, tab_hbm, out_hbm,
               x_v, conn_v, line_v, out_v, *sems):
    _sc_body(x_hbm, conn_hbm, tab_hbm, out_hbm,
             x_v, conn_v, line_v, out_v, *sems)


def kernel(x, conn, tables):
    out = _sc_kernel(x, jnp.transpose(conn, (2, 0, 1)), tables)
    return out.astype(jnp.uint8)


# speedup vs baseline: 1.0971x; 1.0971x over previous
import functools

import jax
import jax.numpy as jnp
from jax import lax
from jax.experimental import pallas as pl
from jax.experimental.pallas import tpu as pltpu
from jax.experimental.pallas import tpu_sc as plsc

_INPUT_BITS = 4096
_NUM_SUB = 4
_BITS_PER_SUB = 12
_TABLE = 1 << _BITS_PER_SUB
_NW = 32
_JPW = _INPUT_BITS // _NW
_L = 16
_NCH = _JPW // _L
_LINE = 128
_P = 8


def _sc_body(x_hbm, conn_hbm, tab_hbm, out_hbm,
             x_v, conn_v, line_v, out_v, *sems):
    wid = lax.axis_index("s") * 2 + lax.axis_index("c")
    base = wid * _JPW

    pltpu.sync_copy(x_hbm, x_v)
    pltpu.sync_copy(conn_hbm.at[:, :, pl.ds(base, _JPW)], conn_v)

    tab2 = tab_hbm.reshape(_NUM_SUB * _INPUT_BITS, _TABLE)
    lane = lax.iota(jnp.int32, _L)
    bvecs = [jnp.full((_L,), b, jnp.int32) for b in range(_BITS_PER_SUB)]
    kmasks = [lane == k for k in range(_L)]

    ones = [jnp.zeros((_L,), jnp.int32) for _ in range(_NCH)]
    pending = []

    def fire(jc, i):
        jloc = jc * _L + lane
        ivec = jnp.full((_L,), i, jnp.int32)
        addr = jnp.zeros((_L,), jnp.int32)
        for b in range(_BITS_PER_SUB):
            bits = plsc.load_gather(conn_v, [bvecs[b], ivec, jloc])
            bit = plsc.load_gather(x_v, [bits])
            addr = addr + (bit << b)
        ahi = addr >> 7
        slot = (jc * _NUM_SUB + i) % _P
        copies = []
        for k in range(_L):
            ahi_k = jnp.max(jnp.where(kmasks[k], ahi, 0))
            row_k = base + (i * _INPUT_BITS + jc * _L + k)
            copies.append(pltpu.async_copy(
                tab2.at[pl.ds(row_k, 1), pl.ds(ahi_k * _LINE, _LINE)],
                line_v.at[pl.ds(slot * _L + k, 1), :], sems[slot]))
        pending.append((jc, i, addr, copies))

    def drain():
        jc, i, addr, copies = pending.pop(0)
        slot = (jc * _NUM_SUB + i) % _P
        pltpu.make_async_copy(
            tab2.at[pl.ds(0, _L), pl.ds(0, _LINE)],
            line_v.at[pl.ds(slot * _L, _L), :], sems[slot]).wait()
        v = plsc.load_gather(
            line_v, [slot * _L + lane, addr & (_LINE - 1)])
        ones[jc] = ones[jc] + jnp.where(v > 0.5, 1, 0).astype(jnp.int32)

    for c in range(_NCH * _NUM_SUB):
        fire(c // _NUM_SUB, c % _NUM_SUB)
        if len(pending) == _P:
            drain()
    while pending:
        drain()

    for jc in range(_NCH):
        out_v[pl.ds(jc * _L, _L)] = (
            jnp.where(ones[jc] > 2, 1, 0).astype(jnp.int32))

    pltpu.sync_copy(out_v, out_hbm.at[pl.ds(base, _JPW)])


@functools.partial(
    pl.kernel,
    out_type=jax.ShapeDtypeStruct((_INPUT_BITS,), jnp.int32),
    mesh=plsc.VectorSubcoreMesh(core_axis_name="c", subcore_axis_name="s"),
    compiler_params=pltpu.CompilerParams(needs_layout_passes=False),
    scratch_types=(
        [pltpu.VMEM((_INPUT_BITS,), jnp.int32),
         pltpu.VMEM((_BITS_PER_SUB, _NUM_SUB, _JPW), jnp.int32),
         pltpu.VMEM((_P * _L, _LINE), jnp.float32),
         pltpu.VMEM((_JPW,), jnp.int32)]
        + [pltpu.SemaphoreType.DMA] * _P),
)
def _sc_kernel(x_hbm, conn_hbm, tab_hbm, out_hbm,
               x_v, conn_v, line_v, out_v, *sems):
    _sc_body(x_hbm, conn_hbm, tab_hbm, out_hbm,
             x_v, conn_v, line_v, out_v, *sems)


def kernel(x, conn, tables):
    out = _sc_kernel(x, jnp.transpose(conn, (2, 0, 1)), tables)
    return out.astype(jnp.uint8)
